# Initial kernel scaffold; baseline (speedup 1.0000x reference)
import jax
import jax.numpy as jnp
from jax.experimental import pallas as pl


def kernel(indices, W_pos, W_neg):
    raise NotImplementedError("write your pallas kernel here")


# speedup vs baseline: 553.5561x; 553.5561x over previous
import functools

import jax
import jax.numpy as jnp
from jax import lax
from jax.experimental import pallas as pl
from jax.experimental.pallas import tpu as pltpu
from jax.experimental.pallas import tpu_sc as plsc

V = 100000
B = 16384
L = 200
NC, NS = 2, 16
NW = NC * NS
ROWS_W = B // NW
CHUNK = 64
NCHUNK = ROWS_W // CHUNK
GROUPS = CHUNK // 16


def _diff_body(p_ref, n_ref, o_ref):
    o_ref[...] = p_ref[...] - n_ref[...]


def _diff_table(wp, wn):
    return pl.pallas_call(
        _diff_body,
        out_shape=jax.ShapeDtypeStruct((V,), jnp.float32),
    )(wp, wn)


@functools.partial(
    pl.kernel,
    out_type=jax.ShapeDtypeStruct((B,), jnp.float32),
    mesh=plsc.VectorSubcoreMesh(core_axis_name="c", subcore_axis_name="s"),
    scratch_types=[
        pltpu.VMEM((V,), jnp.float32),
        pltpu.VMEM((CHUNK, L), jnp.int32),
        pltpu.VMEM((ROWS_W,), jnp.float32),
    ],
    compiler_params=pltpu.CompilerParams(needs_layout_passes=False),
)
def _sc_gather(diff_hbm, idx_hbm, out_hbm, table_v, idx_v, acc_v):
    wid = lax.axis_index("s") * NC + lax.axis_index("c")
    base_row = wid * ROWS_W
    pltpu.sync_copy(diff_hbm, table_v)
    lane = lax.broadcasted_iota(jnp.int32, (16,), 0)
    for ci in range(NCHUNK):
        row0 = base_row + ci * CHUNK
        pltpu.sync_copy(idx_hbm.at[pl.ds(row0, CHUNK), :], idx_v)
        for g in range(GROUPS):
            grow = lane + g * 16

            def body(l, acc, grow=grow):
                col = jnp.full((16,), 0, jnp.int32) + l
                idx16 = plsc.load_gather(idx_v, [grow, col])
                vals = plsc.load_gather(table_v, [idx16])
                return acc + vals

            acc = lax.fori_loop(0, L, body, jnp.zeros((16,), jnp.float32),
                                unroll=8)
            acc_v[pl.ds(ci * CHUNK + g * 16, 16)] = acc
    pltpu.sync_copy(acc_v, out_hbm.at[pl.ds(base_row, ROWS_W)])


def kernel(indices, W_pos, W_neg):
    diff = _diff_table(W_pos.reshape(V), W_neg.reshape(V))
    return _sc_gather(diff, indices.astype(jnp.int32))


# speedup vs baseline: 719.1258x; 1.2991x over previous
import functools

import jax
import jax.numpy as jnp
from jax import lax
from jax.experimental import pallas as pl
from jax.experimental.pallas import tpu as pltpu
from jax.experimental.pallas import tpu_sc as plsc

V = 100000
B = 16384
L = 200
NC, NS = 2, 16
NW = NC * NS
ROWS_W = B // NW
CHUNK = 64
NCHUNK = ROWS_W // CHUNK
GROUPS = CHUNK // 16
CWORDS = CHUNK * L


def _diff_body(p_ref, n_ref, o_ref):
    o_ref[...] = p_ref[...] - n_ref[...]


def _diff_table(wp, wn):
    return pl.pallas_call(
        _diff_body,
        out_shape=jax.ShapeDtypeStruct((V,), jnp.float32),
    )(wp, wn)


@functools.partial(
    pl.kernel,
    out_type=jax.ShapeDtypeStruct((B,), jnp.float32),
    mesh=plsc.VectorSubcoreMesh(core_axis_name="c", subcore_axis_name="s"),
    scratch_types=[
        pltpu.VMEM((V,), jnp.float32),
        pltpu.VMEM((CWORDS,), jnp.int32),
        pltpu.VMEM((CWORDS,), jnp.int32),
        pltpu.VMEM((ROWS_W,), jnp.float32),
        pltpu.SemaphoreType.DMA,
        pltpu.SemaphoreType.DMA,
        pltpu.SemaphoreType.DMA,
    ],
    compiler_params=pltpu.CompilerParams(needs_layout_passes=False),
)
def _sc_gather(diff_hbm, idx_hbm, out_hbm, table_v, idx0_v, idx1_v, acc_v,
               sem_t, sem0, sem1):
    wid = lax.axis_index("s") * NC + lax.axis_index("c")
    base = wid * ROWS_W * L
    tbl_cp = pltpu.async_copy(diff_hbm, table_v, sem_t)
    bufs = (idx0_v, idx1_v)
    sems = (sem0, sem1)
    cps = [pltpu.async_copy(idx_hbm.at[pl.ds(base, CWORDS)], idx0_v, sem0),
           None]
    lane = lax.broadcasted_iota(jnp.int32, (16,), 0)
    tbl_cp.wait()
    for ci in range(NCHUNK):
        cur = ci % 2
        nxt = (ci + 1) % 2
        cps[cur].wait()
        if ci + 1 < NCHUNK:
            cps[nxt] = pltpu.async_copy(
                idx_hbm.at[pl.ds(base + (ci + 1) * CWORDS, CWORDS)],
                bufs[nxt], sems[nxt])
        idx_v = bufs[cur]
        for g in range(GROUPS):
            gbase = (lane + g * 16) * L

            def body(l, acc, gbase=gbase, idx_v=idx_v):
                idx16 = plsc.load_gather(idx_v, [gbase + l])
                vals = plsc.load_gather(table_v, [idx16])
                return acc + vals

            acc = lax.fori_loop(0, L, body, jnp.zeros((16,), jnp.float32),
                                unroll=8)
            acc_v[pl.ds(ci * CHUNK + g * 16, 16)] = acc
    pltpu.sync_copy(acc_v, out_hbm.at[pl.ds(wid * ROWS_W, ROWS_W)])


def kernel(indices, W_pos, W_neg):
    diff = _diff_table(W_pos.reshape(V), W_neg.reshape(V))
    idx_flat = indices.reshape(B * L).astype(jnp.int32)
    return _sc_gather(diff, idx_flat)


# speedup vs baseline: 1236.5954x; 1.7196x over previous
import functools

import jax
import jax.numpy as jnp
from jax import lax
from jax.experimental import pallas as pl
from jax.experimental.pallas import tpu as pltpu
from jax.experimental.pallas import tpu_sc as plsc

V = 100000
B = 16384
L = 200
NC, NS = 2, 16
NW = NC * NS
COLS_W = B // NW
CB = 128
NCB = COLS_W // CB
LH0, LH1 = 104, 96


def _diff_body(p_ref, n_ref, o_ref):
    o_ref[...] = p_ref[...] - n_ref[...]


def _diff_table(wp, wn):
    return pl.pallas_call(
        _diff_body,
        out_shape=jax.ShapeDtypeStruct((V,), jnp.float32),
    )(wp, wn)


@functools.partial(
    pl.kernel,
    out_type=jax.ShapeDtypeStruct((B,), jnp.float32),
    mesh=plsc.VectorSubcoreMesh(core_axis_name="c", subcore_axis_name="s"),
    scratch_types=[
        pltpu.VMEM((V,), jnp.float32),
        pltpu.VMEM((LH0, CB), jnp.int32),
        pltpu.VMEM((LH0, CB), jnp.int32),
        pltpu.VMEM((COLS_W,), jnp.float32),
        pltpu.SemaphoreType.DMA,
        pltpu.SemaphoreType.DMA,
        pltpu.SemaphoreType.DMA,
    ],
    compiler_params=pltpu.CompilerParams(needs_layout_passes=False),
)
def _sc_gather(diff_hbm, idxt_hbm, out_hbm, table_v, idx0_v, idx1_v, acc_v,
               sem_t, sem0, sem1):
    wid = lax.axis_index("s") * NC + lax.axis_index("c")
    col0 = wid * COLS_W
    tbl_cp = pltpu.async_copy(diff_hbm, table_v, sem_t)
    bufs = (idx0_v, idx1_v)
    sems = (sem0, sem1)
    sched = [(cb, h) for cb in range(NCB) for h in range(2)]

    def start(i, buf):
        cb, h = sched[i]
        l0, ln = (0, LH0) if h == 0 else (LH0, LH1)
        return pltpu.async_copy(
            idxt_hbm.at[pl.ds(l0, ln), pl.ds(col0 + cb * CB, CB)],
            buf.at[pl.ds(0, ln), :], sems[i % 2])

    cps = [start(0, idx0_v), None]
    tbl_cp.wait()
    for i, (cb, h) in enumerate(sched):
        cps[i % 2].wait()
        if i + 1 < len(sched):
            cps[(i + 1) % 2] = start(i + 1, bufs[(i + 1) % 2])
        chunk = bufs[i % 2]
        ln = LH0 if h == 0 else LH1
        for cg in range(CB // 16):
            def body(l, acc, chunk=chunk, cg=cg):
                idx16 = chunk[l, pl.ds(cg * 16, 16)]
                vals = plsc.load_gather(table_v, [idx16])
                return acc + vals

            acc = lax.fori_loop(0, ln, body, jnp.zeros((16,), jnp.float32),
                                unroll=8)
            off = cb * CB + cg * 16
            if h == 0:
                acc_v[pl.ds(off, 16)] = acc
            else:
                acc_v[pl.ds(off, 16)] = acc_v[pl.ds(off, 16)] + acc
    pltpu.sync_copy(acc_v, out_hbm.at[pl.ds(col0, COLS_W)])


def kernel(indices, W_pos, W_neg):
    diff = _diff_table(W_pos.reshape(V), W_neg.reshape(V))
    idxt = indices.astype(jnp.int32).T
    return _sc_gather(diff, idxt)


# speedup vs baseline: 1345.4365x; 1.0880x over previous
import functools

import jax
import jax.numpy as jnp
from jax import lax
from jax.experimental import pallas as pl
from jax.experimental.pallas import tpu as pltpu
from jax.experimental.pallas import tpu_sc as plsc

V = 100000
B = 16384
L = 200
NC, NS = 2, 16
NW = NC * NS
COLS_W = B // NW
CB = 128
NCB = COLS_W // CB
LQ = 56
QS = (56, 56, 56, 32)
NQ = len(QS)


def _diff_body(p_ref, n_ref, o_ref):
    o_ref[...] = p_ref[...] - n_ref[...]


def _diff_table(wp, wn):
    return pl.pallas_call(
        _diff_body,
        out_shape=jax.ShapeDtypeStruct((V,), jnp.float32),
    )(wp, wn)


@functools.partial(
    pl.kernel,
    out_type=jax.ShapeDtypeStruct((B,), jnp.float32),
    mesh=plsc.VectorSubcoreMesh(core_axis_name="c", subcore_axis_name="s"),
    scratch_types=[
        pltpu.VMEM((V,), jnp.float32),
        pltpu.VMEM_SHARED((V,), jnp.float32),
        pltpu.VMEM((LQ, CB), jnp.int32),
        pltpu.VMEM((LQ, CB), jnp.int32),
        pltpu.VMEM((COLS_W,), jnp.float32),
        pltpu.SemaphoreType.DMA,
        pltpu.SemaphoreType.DMA,
        pltpu.SemaphoreType.DMA,
    ],
    compiler_params=pltpu.CompilerParams(needs_layout_passes=False),
)
def _sc_gather(diff_hbm, idxt_hbm, out_hbm, table_v, tbl_sh, idx0_v, idx1_v,
               acc_v, sem_t, sem0, sem1):
    sid = lax.axis_index("s")
    wid = sid * NC + lax.axis_index("c")
    col0 = wid * COLS_W
    @pl.when(sid == 0)
    def _():
        pltpu.sync_copy(diff_hbm, tbl_sh)

    bufs = (idx0_v, idx1_v)
    sems = (sem0, sem1)
    sched = [(cb, q) for cb in range(NCB) for q in range(NQ)]

    def start(i, buf):
        cb, q = sched[i]
        l0, ln = LQ * q, QS[q]
        return pltpu.async_copy(
            idxt_hbm.at[pl.ds(l0, ln), pl.ds(col0 + cb * CB, CB)],
            buf.at[pl.ds(0, ln), :], sems[i % 2])

    cps = [start(0, idx0_v), None]
    plsc.subcore_barrier()
    pltpu.async_copy(tbl_sh, table_v, sem_t).wait()
    NG = CB // 16
    for i, (cb, q) in enumerate(sched):
        cps[i % 2].wait()
        if i + 1 < len(sched):
            cps[(i + 1) % 2] = start(i + 1, bufs[(i + 1) % 2])
        chunk = bufs[i % 2]
        ln = QS[q]

        def body(l, accs, chunk=chunk):
            out = []
            for cg in range(NG):
                idx16 = chunk[l, pl.ds(cg * 16, 16)]
                vals = plsc.load_gather(table_v, [idx16])
                out.append(accs[cg] + vals)
            return tuple(out)

        if q == 0:
            accs = (jnp.zeros((16,), jnp.float32),) * NG
        accs = lax.fori_loop(0, ln, body, accs, unroll=4)
        if q == NQ - 1:
            for cg in range(NG):
                acc_v[pl.ds(cb * CB + cg * 16, 16)] = accs[cg]
    pltpu.sync_copy(acc_v, out_hbm.at[pl.ds(col0, COLS_W)])


def kernel(indices, W_pos, W_neg):
    diff = _diff_table(W_pos.reshape(V), W_neg.reshape(V))
    idxt = indices.astype(jnp.int32).T
    return _sc_gather(diff, idxt)


# speedup vs baseline: 1415.6871x; 1.0522x over previous
import functools

import jax
import jax.numpy as jnp
from jax import lax
from jax.experimental import pallas as pl
from jax.experimental.pallas import tpu as pltpu
from jax.experimental.pallas import tpu_sc as plsc

V = 100000
B = 16384
L = 200
NC, NS = 2, 16
NW = NC * NS
COLS_W = B // NW
CB = 128
NCB = COLS_W // CB
LQ = 56
QS = (56, 56, 56, 32)
NQ = len(QS)


def _diff_body(p_ref, n_ref, o_ref):
    o_ref[...] = (p_ref[...] - n_ref[...]).reshape(V)


def _diff_table(wp, wn):
    return pl.pallas_call(
        _diff_body,
        out_shape=jax.ShapeDtypeStruct((V,), jnp.float32),
    )(wp, wn)


@functools.partial(
    pl.kernel,
    out_type=jax.ShapeDtypeStruct((B,), jnp.float32),
    mesh=plsc.VectorSubcoreMesh(core_axis_name="c", subcore_axis_name="s"),
    scratch_types=[
        pltpu.VMEM((V,), jnp.float32),
        pltpu.VMEM_SHARED((V,), jnp.float32),
        pltpu.VMEM((LQ, CB), jnp.int32),
        pltpu.VMEM((LQ, CB), jnp.int32),
        pltpu.VMEM((COLS_W,), jnp.float32),
        pltpu.SemaphoreType.DMA,
        pltpu.SemaphoreType.DMA,
        pltpu.SemaphoreType.DMA,
    ],
    compiler_params=pltpu.CompilerParams(needs_layout_passes=False),
)
def _sc_gather(diff_hbm, idxt_hbm, out_hbm, table_v, tbl_sh, idx0_v, idx1_v,
               acc_v, sem_t, sem0, sem1):
    sid = lax.axis_index("s")
    wid = sid * NC + lax.axis_index("c")
    col0 = wid * COLS_W
    @pl.when(sid == 0)
    def _():
        pltpu.sync_copy(diff_hbm, tbl_sh)

    bufs = (idx0_v, idx1_v)
    sems = (sem0, sem1)
    sched = [(cb, q) for cb in range(NCB) for q in range(NQ)]

    def start(i, buf):
        cb, q = sched[i]
        l0, ln = LQ * q, QS[q]
        return pltpu.async_copy(
            idxt_hbm.at[pl.ds(l0, ln), pl.ds(col0 + cb * CB, CB)],
            buf.at[pl.ds(0, ln), :], sems[i % 2])

    cps = [start(0, idx0_v), None]
    plsc.subcore_barrier()
    pltpu.async_copy(tbl_sh, table_v, sem_t).wait()
    NG = CB // 16
    for i, (cb, q) in enumerate(sched):
        cps[i % 2].wait()
        if i + 1 < len(sched):
            cps[(i + 1) % 2] = start(i + 1, bufs[(i + 1) % 2])
        chunk = bufs[i % 2]
        ln = QS[q]

        def body(l, accs, chunk=chunk):
            out = []
            for cg in range(NG):
                idx16 = chunk[l, pl.ds(cg * 16, 16)]
                vals = plsc.load_gather(table_v, [idx16])
                out.append(accs[cg] + vals)
            return tuple(out)

        if q == 0:
            accs = (jnp.zeros((16,), jnp.float32),) * NG
        accs = lax.fori_loop(0, ln, body, accs, unroll=4)
        if q == NQ - 1:
            for cg in range(NG):
                acc_v[pl.ds(cb * CB + cg * 16, 16)] = accs[cg]
    pltpu.sync_copy(acc_v, out_hbm.at[pl.ds(col0, COLS_W)])


def kernel(indices, W_pos, W_neg):
    diff = _diff_table(W_pos.T, W_neg.T)
    idxt = indices.astype(jnp.int32).T
    return _sc_gather(diff, idxt)
